# baseline (speedup 1.0000x reference)
import jax
import jax.numpy as jnp
from jax.experimental import pallas as pl
from jax.experimental.pallas import tpu as pltpu
from jax.experimental.xla_metadata import set_xla_metadata

_EPS = 1e-20
_N_ITERS = 5
_N_CHUNKS = 4


def _make_body(S, K, D, DL):
    KK = K * K
    KD = K * D

    def _body(latX_ref, seqX_ref, *rest):
        noise_refs = rest[:S]
        ws_ref, bs_ref, wm_ref, bm_ref, ord_ref, stop_ref = rest[S:]

        lat = [latX_ref[dl, 0] for dl in range(DL)]

        a0 = []
        for rj in range(KK):
            acc = lat[0] * ws_ref[0, rj]
            for dl in range(1, DL):
                acc = acc + lat[dl] * ws_ref[dl, rj]
            a0.append(jnp.exp(acc + bs_ref[rj]))

        st = []
        for k in range(K):
            acc = lat[0] * wm_ref[0, k]
            for dl in range(1, DL):
                acc = acc + lat[dl] * wm_ref[dl, k]
            st.append(acc + bm_ref[k])
        m = st[0]
        for k in range(1, K):
            m = jnp.maximum(m, st[k])
        e = [jnp.exp(x - m) for x in st]
        ssum = e[0]
        for k in range(1, K):
            ssum = ssum + e[k]
        rs = 1.0 / ssum
        for k in range(K):
            stop_ref[k, 0] = e[k] * rs

        seqv = [seqX_ref[jd, 0] for jd in range(KD)]

        for s in range(S):
            a = []
            for r in range(K):
                arow = []
                for j in range(K):
                    u = noise_refs[s][K * r + j, 0, 0]
                    denom = _EPS - jnp.log(u + _EPS)
                    arow.append(a0[K * r + j] / denom)
                a.append(arow)
            for _ in range(_N_ITERS):
                for r in range(K):
                    t = a[r][0]
                    for j in range(1, K):
                        t = t + a[r][j]
                    rr = 1.0 / t
                    a[r] = [x * rr for x in a[r]]
                for j in range(K):
                    t = a[0][j]
                    for r in range(1, K):
                        t = t + a[r][j]
                    rc = 1.0 / t
                    for r in range(K):
                        a[r][j] = a[r][j] * rc
            for i in range(K):
                for d in range(D):
                    acc = a[0][i] * seqv[d]
                    for j in range(1, K):
                        acc = acc + a[j][i] * seqv[D * j + d]
                    ord_ref[s, D * i + d, 0] = acc

    return _body


def kernel(latent, seq, noise_u, W_sink, b_sink, W_mask, b_mask):
    B, DL = latent.shape
    _, K, D = seq.shape
    S = noise_u.shape[0] // B
    KK = K * K
    KD = K * D

    nc = _N_CHUNKS if B % (_N_CHUNKS * 1024) == 0 else 1
    Bc = B // nc
    nb = Bc // 1024

    seq2 = seq.reshape(B, KD)
    noise3 = noise_u.reshape(S, B, KK)

    body = _make_body(S, K, D, DL)
    noise_specs = [
        pl.BlockSpec((KK, 1, 1, 8, 128), lambda i, s=s: (0, s, i, 0, 0))
        for s in range(S)
    ]

    ord_pieces = []
    stop_pieces = []
    for c in range(nc):
        sl = slice(c * Bc, (c + 1) * Bc)
        with set_xla_metadata(_scheduling_group_id=max(c - 1, 0)):
            latX = latent[sl].T.reshape(DL, nb, 8, 128)
            seqX = seq2[sl].T.reshape(KD, nb, 8, 128)
            noiseX = noise3[:, sl].transpose(2, 0, 1).reshape(
                KK, S, nb, 8, 128)

        with set_xla_metadata(_scheduling_group_id=c):
            ordX, stopX = pl.pallas_call(
                    body,
                    grid=(nb,),
                in_specs=[
                    pl.BlockSpec((DL, 1, 8, 128), lambda i: (0, i, 0, 0)),
                    pl.BlockSpec((KD, 1, 8, 128), lambda i: (0, i, 0, 0)),
                    *noise_specs,
                    pl.BlockSpec(memory_space=pltpu.SMEM),
                    pl.BlockSpec(memory_space=pltpu.SMEM),
                    pl.BlockSpec(memory_space=pltpu.SMEM),
                    pl.BlockSpec(memory_space=pltpu.SMEM),
                ],
                out_specs=[
                    pl.BlockSpec((S, KD, 1, 8, 128), lambda i: (0, 0, i, 0, 0)),
                    pl.BlockSpec((K, 1, 8, 128), lambda i: (0, i, 0, 0)),
                ],
                out_shape=[
                    jax.ShapeDtypeStruct((S, KD, nb, 8, 128), jnp.float32),
                    jax.ShapeDtypeStruct((K, nb, 8, 128), jnp.float32),
                ],
                compiler_params=pltpu.CompilerParams(
                    dimension_semantics=("parallel",),
                ),
                name="sinkhorn_net",
            )(latX, seqX, *([noiseX] * S), W_sink, b_sink, W_mask, b_mask)

        with set_xla_metadata(_scheduling_group_id=min(c + 1, nc - 1)):
            ord_pieces.append(
                ordX.reshape(S, KD, Bc).transpose(0, 2, 1).reshape(
                    S, Bc, K, D))
            stop_pieces.append(stopX.reshape(K, Bc).T)

    ordered = jnp.concatenate(ord_pieces, axis=1).reshape(S * B, K, D)
    stopping = jnp.concatenate(stop_pieces, axis=0)
    return ordered, stopping


# speedup vs baseline: 1.0006x; 1.0006x over previous
import jax
import jax.numpy as jnp
from jax.experimental import pallas as pl
from jax.experimental.pallas import tpu as pltpu
from jax.experimental.xla_metadata import set_xla_metadata

_EPS = 1e-20
_N_ITERS = 5
_N_CHUNKS = 4


def _make_body(S, K, D, DL):
    KK = K * K
    KD = K * D

    def _body(latX_ref, seqX_ref, *rest):
        noise_refs = rest[:S]
        ws_ref, bs_ref, wm_ref, bm_ref, ord_ref, stop_ref = rest[S:]

        lat = [latX_ref[dl, 0] for dl in range(DL)]

        a0 = []
        for rj in range(KK):
            acc = lat[0] * ws_ref[0, rj]
            for dl in range(1, DL):
                acc = acc + lat[dl] * ws_ref[dl, rj]
            a0.append(jnp.exp(acc + bs_ref[rj]))

        st = []
        for k in range(K):
            acc = lat[0] * wm_ref[0, k]
            for dl in range(1, DL):
                acc = acc + lat[dl] * wm_ref[dl, k]
            st.append(acc + bm_ref[k])
        m = st[0]
        for k in range(1, K):
            m = jnp.maximum(m, st[k])
        e = [jnp.exp(x - m) for x in st]
        ssum = e[0]
        for k in range(1, K):
            ssum = ssum + e[k]
        rs = 1.0 / ssum
        for k in range(K):
            stop_ref[k, 0] = e[k] * rs

        seqv = [seqX_ref[jd, 0] for jd in range(KD)]

        for s in range(S):
            a = []
            for r in range(K):
                arow = []
                for j in range(K):
                    u = noise_refs[s][K * r + j, 0, 0]
                    denom = _EPS - jnp.log(u + _EPS)
                    arow.append(a0[K * r + j] / denom)
                a.append(arow)
            for _ in range(_N_ITERS):
                for r in range(K):
                    t = a[r][0]
                    for j in range(1, K):
                        t = t + a[r][j]
                    rr = 1.0 / t
                    a[r] = [x * rr for x in a[r]]
                for j in range(K):
                    t = a[0][j]
                    for r in range(1, K):
                        t = t + a[r][j]
                    rc = 1.0 / t
                    for r in range(K):
                        a[r][j] = a[r][j] * rc
            for i in range(K):
                for d in range(D):
                    acc = a[0][i] * seqv[d]
                    for j in range(1, K):
                        acc = acc + a[j][i] * seqv[D * j + d]
                    ord_ref[s, D * i + d, 0] = acc

    return _body


def kernel(latent, seq, noise_u, W_sink, b_sink, W_mask, b_mask):
    B, DL = latent.shape
    _, K, D = seq.shape
    S = noise_u.shape[0] // B
    KK = K * K
    KD = K * D

    nc = _N_CHUNKS if B % (_N_CHUNKS * 1024) == 0 else 1
    Bc = B // nc
    nb = Bc // 1024

    seq2 = seq.reshape(B, KD)
    noise3 = noise_u.reshape(S, B, KK)

    body = _make_body(S, K, D, DL)
    noise_specs = [
        pl.BlockSpec((KK, 1, 1, 8, 128), lambda i, s=s: (0, s, i, 0, 0))
        for s in range(S)
    ]

    ord_pieces = []
    stop_pieces = []
    for c in range(nc):
        sl = slice(c * Bc, (c + 1) * Bc)
        with set_xla_metadata(_scheduling_group_id=max(c - 1, 0)):
            latX = latent[sl].T.reshape(DL, nb, 8, 128)
            seqX = seq2[sl].T.reshape(KD, nb, 8, 128)
            noiseX = noise3[:, sl].transpose(2, 0, 1).reshape(
                KK, S, nb, 8, 128)

        with set_xla_metadata(_scheduling_group_id=c):
            ordX, stopX = pl.pallas_call(
                    body,
                    grid=(nb,),
                in_specs=[
                    pl.BlockSpec((DL, 1, 8, 128), lambda i: (0, i, 0, 0)),
                    pl.BlockSpec((KD, 1, 8, 128), lambda i: (0, i, 0, 0)),
                    *noise_specs,
                    pl.BlockSpec(memory_space=pltpu.SMEM),
                    pl.BlockSpec(memory_space=pltpu.SMEM),
                    pl.BlockSpec(memory_space=pltpu.SMEM),
                    pl.BlockSpec(memory_space=pltpu.SMEM),
                ],
                out_specs=[
                    pl.BlockSpec((S, KD, 1, 8, 128), lambda i: (0, 0, i, 0, 0)),
                    pl.BlockSpec((K, 1, 8, 128), lambda i: (0, i, 0, 0)),
                ],
                out_shape=[
                    jax.ShapeDtypeStruct((S, KD, nb, 8, 128), jnp.float32),
                    jax.ShapeDtypeStruct((K, nb, 8, 128), jnp.float32),
                ],
                compiler_params=pltpu.CompilerParams(
                    dimension_semantics=("parallel",),
                    allow_input_fusion=[True] * (S + 6),
                ),
                name="sinkhorn_net",
            )(latX, seqX, *([noiseX] * S), W_sink, b_sink, W_mask, b_mask)

        with set_xla_metadata(_scheduling_group_id=min(c + 1, nc - 1)):
            ord_pieces.append(
                ordX.reshape(S, KD, Bc).transpose(0, 2, 1).reshape(
                    S, Bc, K, D))
            stop_pieces.append(stopX.reshape(K, Bc).T)

    ordered = jnp.concatenate(ord_pieces, axis=1).reshape(S * B, K, D)
    stopping = jnp.concatenate(stop_pieces, axis=0)
    return ordered, stopping


# speedup vs baseline: 1.1587x; 1.1580x over previous
import jax
import jax.numpy as jnp
from jax.experimental import pallas as pl
from jax.experimental.pallas import tpu as pltpu

_EPS = 1e-20
_N_ITERS = 5
_N_CHUNKS = 4


def _make_body(S, K, D, DL):
    KK = K * K
    KD = K * D

    def _body(latX_ref, seqX_ref, *rest):
        noise_refs = rest[:S]
        ws_ref, bs_ref, wm_ref, bm_ref, ord_ref, stop_ref = rest[S:]

        lat = [latX_ref[dl, 0, 0] for dl in range(DL)]

        a0 = []
        for rj in range(KK):
            acc = lat[0] * ws_ref[0, rj]
            for dl in range(1, DL):
                acc = acc + lat[dl] * ws_ref[dl, rj]
            a0.append(jnp.exp(acc + bs_ref[rj]))

        st = []
        for k in range(K):
            acc = lat[0] * wm_ref[0, k]
            for dl in range(1, DL):
                acc = acc + lat[dl] * wm_ref[dl, k]
            st.append(acc + bm_ref[k])
        m = st[0]
        for k in range(1, K):
            m = jnp.maximum(m, st[k])
        e = [jnp.exp(x - m) for x in st]
        ssum = e[0]
        for k in range(1, K):
            ssum = ssum + e[k]
        rs = 1.0 / ssum
        for k in range(K):
            stop_ref[k, 0] = e[k] * rs

        seqv = [seqX_ref[jd, 0, 0] for jd in range(KD)]

        for s in range(S):
            a = []
            for r in range(K):
                arow = []
                for j in range(K):
                    u = noise_refs[s][K * r + j, 0, 0, 0]
                    denom = _EPS - jnp.log(u + _EPS)
                    arow.append(a0[K * r + j] / denom)
                a.append(arow)
            for _ in range(_N_ITERS):
                for r in range(K):
                    t = a[r][0]
                    for j in range(1, K):
                        t = t + a[r][j]
                    rr = 1.0 / t
                    a[r] = [x * rr for x in a[r]]
                for j in range(K):
                    t = a[0][j]
                    for r in range(1, K):
                        t = t + a[r][j]
                    rc = 1.0 / t
                    for r in range(K):
                        a[r][j] = a[r][j] * rc
            for i in range(K):
                for d in range(D):
                    acc = a[0][i] * seqv[d]
                    for j in range(1, K):
                        acc = acc + a[j][i] * seqv[D * j + d]
                    ord_ref[s, D * i + d, 0] = acc

    return _body


def kernel(latent, seq, noise_u, W_sink, b_sink, W_mask, b_mask):
    B, DL = latent.shape
    _, K, D = seq.shape
    S = noise_u.shape[0] // B
    KK = K * K
    KD = K * D

    nc = _N_CHUNKS if B % (_N_CHUNKS * 1024) == 0 else 1
    Bc = B // nc
    nb = Bc // 1024

    latX = latent.T.reshape(DL, nc, nb, 8, 128)
    seqX = seq.reshape(B, KD).T.reshape(KD, nc, nb, 8, 128)
    noiseX = noise_u.reshape(S * B, KK).T.reshape(KK, S, nc, nb, 8, 128)

    body = _make_body(S, K, D, DL)

    ord_pieces = []
    stop_pieces = []
    for c in range(nc):
        noise_specs = [
            pl.BlockSpec((KK, 1, 1, 1, 8, 128),
                         lambda i, s=s, c=c: (0, s, c, i, 0, 0))
            for s in range(S)
        ]
        ordX, stopX = pl.pallas_call(
            body,
            grid=(nb,),
            in_specs=[
                pl.BlockSpec((DL, 1, 1, 8, 128),
                             lambda i, c=c: (0, c, i, 0, 0)),
                pl.BlockSpec((KD, 1, 1, 8, 128),
                             lambda i, c=c: (0, c, i, 0, 0)),
                *noise_specs,
                pl.BlockSpec(memory_space=pltpu.SMEM),
                pl.BlockSpec(memory_space=pltpu.SMEM),
                pl.BlockSpec(memory_space=pltpu.SMEM),
                pl.BlockSpec(memory_space=pltpu.SMEM),
            ],
            out_specs=[
                pl.BlockSpec((S, KD, 1, 8, 128), lambda i: (0, 0, i, 0, 0)),
                pl.BlockSpec((K, 1, 8, 128), lambda i: (0, i, 0, 0)),
            ],
            out_shape=[
                jax.ShapeDtypeStruct((S, KD, nb, 8, 128), jnp.float32),
                jax.ShapeDtypeStruct((K, nb, 8, 128), jnp.float32),
            ],
            compiler_params=pltpu.CompilerParams(
                dimension_semantics=("parallel",),
            ),
            name="sinkhorn_net",
        )(latX, seqX, *([noiseX] * S), W_sink, b_sink, W_mask, b_mask)

        ord_pieces.append(
            ordX.reshape(S, KD, Bc).transpose(0, 2, 1).reshape(
                S, Bc, K, D))
        stop_pieces.append(stopX.reshape(K, Bc).T)

    ordered = jnp.concatenate(ord_pieces, axis=1).reshape(S * B, K, D)
    stopping = jnp.concatenate(stop_pieces, axis=0)
    return ordered, stopping
